# baseline (speedup 1.0000x reference)
import math

import jax
import jax.numpy as jnp
import numpy as np
from jax.experimental import pallas as pl
from jax.experimental.pallas import tpu as pltpu

NLAT = 512
NLON = 1024
LMAX = 50
MMAX = 50
MPAD = 64
NPTS = 512
TR = 32
TC = 128
NTR = NLAT // TR
NTC = NLON // TC
SCALE = math.pi / 512.0
UNROLL = 16


def _cc_weights(n):
    N = n - 1
    theta = np.pi * np.arange(n) / N
    w = np.zeros(n)
    v = np.ones(n - 2)
    if N % 2 == 0:
        w0 = 1.0 / (N * N - 1)
        for k in range(1, N // 2):
            v -= 2.0 * np.cos(2 * k * theta[1:-1]) / (4 * k * k - 1)
        v -= np.cos(N * theta[1:-1]) / (N * N - 1)
    else:
        w0 = 1.0 / (N * N)
        for k in range(1, (N - 1) // 2 + 1):
            v -= 2.0 * np.cos(2 * k * theta[1:-1]) / (4 * k * k - 1)
    w[0] = w0
    w[-1] = w0
    w[1:-1] = 2.0 * v / N
    return w


def _legendre(lmax, mmax, x):
    nx = x.shape[0]
    P = np.zeros((lmax, mmax, nx))
    P[0, 0] = np.sqrt(1.0 / (4.0 * np.pi))
    s = np.sqrt(np.maximum(0.0, 1.0 - x * x))
    for m in range(1, mmax):
        P[m, m] = -np.sqrt((2.0 * m + 1.0) / (2.0 * m)) * s * P[m - 1, m - 1]
    for m in range(0, mmax):
        if m + 1 < lmax:
            P[m + 1, m] = np.sqrt(2.0 * m + 3.0) * x * P[m, m]
        for l in range(m + 2, lmax):
            a = np.sqrt((4.0 * l * l - 1.0) / (l * l - m * m))
            b = np.sqrt(((l - 1.0) ** 2 - m * m) / (4.0 * (l - 1.0) ** 2 - 1.0))
            P[l, m] = a * (x * P[l - 1, m] - b * P[l - 2, m])
    return P


def _build_consts():
    theta = np.pi * np.arange(NLAT) / (NLAT - 1)
    cost = np.cos(theta)
    wq = _cc_weights(NLAT)
    pct = (_legendre(LMAX, MMAX, cost) * wq[None, None, :])
    A = np.zeros((LMAX * MPAD, NLAT), np.float32)
    A.reshape(LMAX, MPAD, NLAT)[:, :MMAX, :] = pct
    n = np.arange(NLON)[:, None]
    m = np.arange(MPAD)[None, :]
    C = np.cos(2.0 * np.pi * m * n / NLON) * (2.0 * np.pi / NLON)
    C[:, MMAX:] = 0.0
    return A.astype(np.float32), C.astype(np.float32)


_A_NP, _C_NP = _build_consts()

_CX_NP = ((np.arange(NTR) * TR + (TR - 1) / 2.0) * SCALE).astype(np.float32)
_CY_NP = ((np.arange(NTC) * TC + (TC - 1) / 2.0) * SCALE - math.pi).astype(np.float32)
_HX = (TR - 1) / 2.0 * SCALE
_HY = (TC - 1) / 2.0 * SCALE


def _to_spherical(coords):
    x = coords[..., 0]
    y = coords[..., 1]
    z = coords[..., 2]
    r = jnp.sqrt(x * x + y * y + z * z)
    phi1 = jnp.arccos(jnp.clip(x / r, -1.0, 1.0))
    azn = jnp.sqrt(y * y + z * z)
    a = jnp.arccos(jnp.clip(y / azn, -1.0, 1.0))
    phi2 = a + (2.0 * math.pi - 2.0 * a) * (z < 0) - math.pi
    return r, jnp.stack([phi1, phi2], axis=-1)


def _tile_metadata(pts):
    px = pts[:, :, 0][:, None, None, :]
    py = pts[:, :, 1][:, None, None, :]
    cx = jnp.asarray(_CX_NP)[None, :, None, None]
    cy = jnp.asarray(_CY_NP)[None, None, :, None]
    ax = jnp.abs(px - cx)
    ay = jnp.abs(py - cy)
    lbx = jnp.maximum(ax - _HX, 0.0)
    lby = jnp.maximum(ay - _HY, 0.0)
    lb = lbx * lbx + lby * lby
    ubx = ax + _HX
    uby = ay + _HY
    ub = ubx * ubx + uby * uby
    ub3 = -jax.lax.top_k(-ub, 3)[0][..., 2:3]
    keep = lb <= ub3
    key = jnp.where(keep, 0, 1024) + jnp.arange(NPTS, dtype=jnp.int32)
    order = jnp.argsort(key, axis=-1).astype(jnp.int32)
    cnt = jnp.sum(keep, axis=-1, dtype=jnp.int32)
    cntu = jnp.minimum((cnt + UNROLL - 1) // UNROLL, NPTS // UNROLL)
    return order, cntu


def _nn_interp_kernel(spx_ref, spy_ref, sr_ref, cnt_ref, out_ref):
    tr = pl.program_id(1)
    tc = pl.program_id(2)
    rowi = jax.lax.broadcasted_iota(jnp.int32, (TR, TC), 0).astype(jnp.float32)
    coli = jax.lax.broadcasted_iota(jnp.int32, (TR, TC), 1).astype(jnp.float32)
    gx = (rowi + (tr * TR).astype(jnp.float32)) * SCALE
    gy = (coli + (tc * TC).astype(jnp.float32)) * SCALE - math.pi

    big = jnp.full((TR, TC), 1e30, jnp.float32)
    zero = jnp.zeros((TR, TC), jnp.float32)

    def insert(c, carry):
        m1, m2, m3, r1, r2, r3 = carry
        px = spx_ref[0, 0, 0, c]
        py = spy_ref[0, 0, 0, c]
        rv = sr_ref[0, 0, 0, c]
        dx = gx - px
        dy = gy - py
        d = dx * dx + dy * dy
        c1 = d < m1
        nm1 = jnp.minimum(m1, d)
        t = jnp.maximum(m1, d)
        nr1 = jnp.where(c1, rv, r1)
        tr = jnp.where(c1, r1, rv)
        c2 = t < m2
        nm2 = jnp.minimum(m2, t)
        t2 = jnp.maximum(m2, t)
        nr2 = jnp.where(c2, tr, r2)
        tr2 = jnp.where(c2, r2, tr)
        c3 = t2 < m3
        nm3 = jnp.minimum(m3, t2)
        nr3 = jnp.where(c3, tr2, r3)
        return nm1, nm2, nm3, nr1, nr2, nr3

    def body(k, carry):
        base = k * UNROLL
        for u in range(UNROLL):
            carry = insert(base + u, carry)
        return carry

    m1, m2, m3, r1, r2, r3 = jax.lax.fori_loop(
        0, cnt_ref[0, 0, 0, 0], body, (big, big, big, zero, zero, zero))
    s = m1 + m2 + m3
    out_ref[0] = (r1 * m1 + r2 * m2 + r3 * m3) / s


def _sht_loss_kernel(interp_ref, c_ref, a_ref, out_ref):
    diff = interp_ref[0] - interp_ref[1]
    xc = jnp.dot(diff, c_ref[...], preferred_element_type=jnp.float32)
    y = jnp.dot(a_ref[...], xc, preferred_element_type=jnp.float32)
    row_m = jax.lax.broadcasted_iota(jnp.int32, y.shape, 0) % MPAD
    col_m = jax.lax.broadcasted_iota(jnp.int32, y.shape, 1)
    v = jnp.where(row_m == col_m, y, 0.0)
    out_ref[0, 0] = jnp.sum(v * v) / float(LMAX * MMAX)


def kernel(pred, target):
    rp, sp = _to_spherical(pred)
    rt, st = _to_spherical(target)
    pts = jnp.concatenate([sp, st], axis=0)
    rads = jnp.concatenate([rp, rt], axis=0)
    order, cntu = _tile_metadata(pts)

    ordf = order.reshape(2, NTR * NTC, NPTS)
    spx = jnp.take_along_axis(pts[:, None, :, 0], ordf, axis=-1)
    spy = jnp.take_along_axis(pts[:, None, :, 1], ordf, axis=-1)
    sr = jnp.take_along_axis(rads[:, None, :], ordf, axis=-1)
    smem_shape = (2, NTR * NTC, 1, NPTS)
    spx = spx.reshape(smem_shape)
    spy = spy.reshape(smem_shape)
    sr = sr.reshape(smem_shape)
    cntu = cntu.reshape(2, NTR * NTC, 1, 1)

    list_spec = pl.BlockSpec((1, 1, 1, NPTS),
                             lambda cl, tr, tc: (cl, tr * NTC + tc, 0, 0),
                             memory_space=pltpu.SMEM)
    interp = pl.pallas_call(
        _nn_interp_kernel,
        grid=(2, NTR, NTC),
        in_specs=[
            list_spec,
            list_spec,
            list_spec,
            pl.BlockSpec((1, 1, 1, 1),
                         lambda cl, tr, tc: (cl, tr * NTC + tc, 0, 0),
                         memory_space=pltpu.SMEM),
        ],
        out_specs=pl.BlockSpec((1, TR, TC), lambda cl, tr, tc: (cl, tr, tc)),
        out_shape=jax.ShapeDtypeStruct((2, NLAT, NLON), jnp.float32),
        compiler_params=pltpu.CompilerParams(
            dimension_semantics=("arbitrary", "arbitrary", "arbitrary")),
    )(spx, spy, sr, cntu)

    loss = pl.pallas_call(
        _sht_loss_kernel,
        in_specs=[
            pl.BlockSpec(memory_space=pltpu.VMEM),
            pl.BlockSpec(memory_space=pltpu.VMEM),
            pl.BlockSpec(memory_space=pltpu.VMEM),
        ],
        out_specs=pl.BlockSpec(memory_space=pltpu.SMEM),
        out_shape=jax.ShapeDtypeStruct((1, 1), jnp.float32),
    )(interp, jnp.asarray(_C_NP), jnp.asarray(_A_NP))

    return loss[0, 0]


# speedup vs baseline: 11.5536x; 11.5536x over previous
import math

import jax
import jax.numpy as jnp
import numpy as np
from jax.experimental import pallas as pl
from jax.experimental.pallas import tpu as pltpu

NLAT = 512
NLON = 1024
LMAX = 50
MMAX = 50
MPAD = 64
NPTS = 512
TR = 32
TC = 128
NTR = NLAT // TR
NTC = NLON // TC
SCALE = math.pi / 512.0
UNROLL = 16


def _cc_weights(n):
    N = n - 1
    theta = np.pi * np.arange(n) / N
    w = np.zeros(n)
    v = np.ones(n - 2)
    if N % 2 == 0:
        w0 = 1.0 / (N * N - 1)
        for k in range(1, N // 2):
            v -= 2.0 * np.cos(2 * k * theta[1:-1]) / (4 * k * k - 1)
        v -= np.cos(N * theta[1:-1]) / (N * N - 1)
    else:
        w0 = 1.0 / (N * N)
        for k in range(1, (N - 1) // 2 + 1):
            v -= 2.0 * np.cos(2 * k * theta[1:-1]) / (4 * k * k - 1)
    w[0] = w0
    w[-1] = w0
    w[1:-1] = 2.0 * v / N
    return w


def _legendre(lmax, mmax, x):
    nx = x.shape[0]
    P = np.zeros((lmax, mmax, nx))
    P[0, 0] = np.sqrt(1.0 / (4.0 * np.pi))
    s = np.sqrt(np.maximum(0.0, 1.0 - x * x))
    for m in range(1, mmax):
        P[m, m] = -np.sqrt((2.0 * m + 1.0) / (2.0 * m)) * s * P[m - 1, m - 1]
    for m in range(0, mmax):
        if m + 1 < lmax:
            P[m + 1, m] = np.sqrt(2.0 * m + 3.0) * x * P[m, m]
        for l in range(m + 2, lmax):
            a = np.sqrt((4.0 * l * l - 1.0) / (l * l - m * m))
            b = np.sqrt(((l - 1.0) ** 2 - m * m) / (4.0 * (l - 1.0) ** 2 - 1.0))
            P[l, m] = a * (x * P[l - 1, m] - b * P[l - 2, m])
    return P


def _build_consts():
    theta = np.pi * np.arange(NLAT) / (NLAT - 1)
    cost = np.cos(theta)
    wq = _cc_weights(NLAT)
    pct = (_legendre(LMAX, MMAX, cost) * wq[None, None, :])
    A = np.zeros((LMAX * MPAD, NLAT), np.float32)
    A.reshape(LMAX, MPAD, NLAT)[:, :MMAX, :] = pct
    n = np.arange(NLON)[:, None]
    m = np.arange(MPAD)[None, :]
    C = np.cos(2.0 * np.pi * m * n / NLON) * (2.0 * np.pi / NLON)
    C[:, MMAX:] = 0.0
    return A.astype(np.float32), C.astype(np.float32)


_A_NP, _C_NP = _build_consts()

_CX_NP = ((np.arange(NTR) * TR + (TR - 1) / 2.0) * SCALE).astype(np.float32)
_CY_NP = ((np.arange(NTC) * TC + (TC - 1) / 2.0) * SCALE - math.pi).astype(np.float32)
_HX = (TR - 1) / 2.0 * SCALE
_HY = (TC - 1) / 2.0 * SCALE


def _to_spherical(coords):
    x = coords[..., 0]
    y = coords[..., 1]
    z = coords[..., 2]
    r = jnp.sqrt(x * x + y * y + z * z)
    phi1 = jnp.arccos(jnp.clip(x / r, -1.0, 1.0))
    azn = jnp.sqrt(y * y + z * z)
    a = jnp.arccos(jnp.clip(y / azn, -1.0, 1.0))
    phi2 = a + (2.0 * math.pi - 2.0 * a) * (z < 0) - math.pi
    return r, jnp.stack([phi1, phi2], axis=-1)


def _tile_metadata(pts, rads):
    px = pts[:, :, 0][:, None, None, :]
    py = pts[:, :, 1][:, None, None, :]
    cx = jnp.asarray(_CX_NP)[None, :, None, None]
    cy = jnp.asarray(_CY_NP)[None, None, :, None]
    ax = jnp.abs(px - cx)
    ay = jnp.abs(py - cy)
    lbx = jnp.maximum(ax - _HX, 0.0)
    lby = jnp.maximum(ay - _HY, 0.0)
    lb = lbx * lbx + lby * lby
    ubx = ax + _HX
    uby = ay + _HY
    ub = ubx * ubx + uby * uby
    ub3 = -jax.lax.top_k(-ub, 3)[0][..., 2:3]
    keep = lb <= ub3
    key = jnp.where(keep, 0, 1024) + jnp.arange(NPTS, dtype=jnp.int32)
    shape = key.shape
    bx = jnp.broadcast_to(px, shape)
    by = jnp.broadcast_to(py, shape)
    br = jnp.broadcast_to(rads[:, None, None, :], shape)
    _, spx, spy, sr = jax.lax.sort((key, bx, by, br), dimension=-1, num_keys=1)
    cnt = jnp.sum(keep, axis=-1, dtype=jnp.int32)
    cntu = jnp.minimum((cnt + UNROLL - 1) // UNROLL, NPTS // UNROLL)
    return spx, spy, sr, cntu


def _nn_interp_kernel(spx_ref, spy_ref, sr_ref, cnt_ref, out_ref):
    tr = pl.program_id(1)
    tc = pl.program_id(2)
    rowi = jax.lax.broadcasted_iota(jnp.int32, (TR, TC), 0).astype(jnp.float32)
    coli = jax.lax.broadcasted_iota(jnp.int32, (TR, TC), 1).astype(jnp.float32)
    gx = (rowi + (tr * TR).astype(jnp.float32)) * SCALE
    gy = (coli + (tc * TC).astype(jnp.float32)) * SCALE - math.pi

    big = jnp.full((TR, TC), 1e30, jnp.float32)
    zero = jnp.zeros((TR, TC), jnp.float32)

    def insert(c, carry):
        m1, m2, m3, r1, r2, r3 = carry
        px = spx_ref[0, 0, 0, c]
        py = spy_ref[0, 0, 0, c]
        rv = sr_ref[0, 0, 0, c]
        dx = gx - px
        dy = gy - py
        d = dx * dx + dy * dy
        c1 = d < m1
        nm1 = jnp.minimum(m1, d)
        t = jnp.maximum(m1, d)
        nr1 = jnp.where(c1, rv, r1)
        tr = jnp.where(c1, r1, rv)
        c2 = t < m2
        nm2 = jnp.minimum(m2, t)
        t2 = jnp.maximum(m2, t)
        nr2 = jnp.where(c2, tr, r2)
        tr2 = jnp.where(c2, r2, tr)
        c3 = t2 < m3
        nm3 = jnp.minimum(m3, t2)
        nr3 = jnp.where(c3, tr2, r3)
        return nm1, nm2, nm3, nr1, nr2, nr3

    def body(k, carry):
        base = k * UNROLL
        for u in range(UNROLL):
            carry = insert(base + u, carry)
        return carry

    m1, m2, m3, r1, r2, r3 = jax.lax.fori_loop(
        0, cnt_ref[0, 0, 0, 0], body, (big, big, big, zero, zero, zero))
    s = m1 + m2 + m3
    out_ref[0] = (r1 * m1 + r2 * m2 + r3 * m3) / s


def _sht_loss_kernel(interp_ref, c_ref, a_ref, out_ref):
    diff = interp_ref[0] - interp_ref[1]
    xc = jnp.dot(diff, c_ref[...], preferred_element_type=jnp.float32)
    y = jnp.dot(a_ref[...], xc, preferred_element_type=jnp.float32)
    row_m = jax.lax.broadcasted_iota(jnp.int32, y.shape, 0) % MPAD
    col_m = jax.lax.broadcasted_iota(jnp.int32, y.shape, 1)
    v = jnp.where(row_m == col_m, y, 0.0)
    out_ref[0, 0] = jnp.sum(v * v) / float(LMAX * MMAX)


def kernel(pred, target):
    rp, sp = _to_spherical(pred)
    rt, st = _to_spherical(target)
    pts = jnp.concatenate([sp, st], axis=0)
    rads = jnp.concatenate([rp, rt], axis=0)
    spx, spy, sr, cntu = _tile_metadata(pts, rads)
    smem_shape = (2, NTR * NTC, 1, NPTS)
    spx = spx.reshape(smem_shape)
    spy = spy.reshape(smem_shape)
    sr = sr.reshape(smem_shape)
    cntu = cntu.reshape(2, NTR * NTC, 1, 1)

    list_spec = pl.BlockSpec((1, 1, 1, NPTS),
                             lambda cl, tr, tc: (cl, tr * NTC + tc, 0, 0),
                             memory_space=pltpu.SMEM)
    interp = pl.pallas_call(
        _nn_interp_kernel,
        grid=(2, NTR, NTC),
        in_specs=[
            list_spec,
            list_spec,
            list_spec,
            pl.BlockSpec((1, 1, 1, 1),
                         lambda cl, tr, tc: (cl, tr * NTC + tc, 0, 0),
                         memory_space=pltpu.SMEM),
        ],
        out_specs=pl.BlockSpec((1, TR, TC), lambda cl, tr, tc: (cl, tr, tc)),
        out_shape=jax.ShapeDtypeStruct((2, NLAT, NLON), jnp.float32),
        compiler_params=pltpu.CompilerParams(
            dimension_semantics=("arbitrary", "arbitrary", "arbitrary")),
    )(spx, spy, sr, cntu)

    loss = pl.pallas_call(
        _sht_loss_kernel,
        in_specs=[
            pl.BlockSpec(memory_space=pltpu.VMEM),
            pl.BlockSpec(memory_space=pltpu.VMEM),
            pl.BlockSpec(memory_space=pltpu.VMEM),
        ],
        out_specs=pl.BlockSpec(memory_space=pltpu.SMEM),
        out_shape=jax.ShapeDtypeStruct((1, 1), jnp.float32),
    )(interp, jnp.asarray(_C_NP), jnp.asarray(_A_NP))

    return loss[0, 0]


# speedup vs baseline: 15.4713x; 1.3391x over previous
import math

import jax
import jax.numpy as jnp
import numpy as np
from jax.experimental import pallas as pl
from jax.experimental.pallas import tpu as pltpu

NLAT = 512
NLON = 1024
LMAX = 50
MMAX = 50
MPAD = 64
NPTS = 512
TR = 32
TC = 128
NTR = NLAT // TR
NTC = NLON // TC
SCALE = math.pi / 512.0
UNROLL = 16


def _cc_weights(n):
    N = n - 1
    theta = np.pi * np.arange(n) / N
    w = np.zeros(n)
    v = np.ones(n - 2)
    if N % 2 == 0:
        w0 = 1.0 / (N * N - 1)
        for k in range(1, N // 2):
            v -= 2.0 * np.cos(2 * k * theta[1:-1]) / (4 * k * k - 1)
        v -= np.cos(N * theta[1:-1]) / (N * N - 1)
    else:
        w0 = 1.0 / (N * N)
        for k in range(1, (N - 1) // 2 + 1):
            v -= 2.0 * np.cos(2 * k * theta[1:-1]) / (4 * k * k - 1)
    w[0] = w0
    w[-1] = w0
    w[1:-1] = 2.0 * v / N
    return w


def _legendre(lmax, mmax, x):
    nx = x.shape[0]
    P = np.zeros((lmax, mmax, nx))
    P[0, 0] = np.sqrt(1.0 / (4.0 * np.pi))
    s = np.sqrt(np.maximum(0.0, 1.0 - x * x))
    for m in range(1, mmax):
        P[m, m] = -np.sqrt((2.0 * m + 1.0) / (2.0 * m)) * s * P[m - 1, m - 1]
    for m in range(0, mmax):
        if m + 1 < lmax:
            P[m + 1, m] = np.sqrt(2.0 * m + 3.0) * x * P[m, m]
        for l in range(m + 2, lmax):
            a = np.sqrt((4.0 * l * l - 1.0) / (l * l - m * m))
            b = np.sqrt(((l - 1.0) ** 2 - m * m) / (4.0 * (l - 1.0) ** 2 - 1.0))
            P[l, m] = a * (x * P[l - 1, m] - b * P[l - 2, m])
    return P


def _build_consts():
    theta = np.pi * np.arange(NLAT) / (NLAT - 1)
    cost = np.cos(theta)
    wq = _cc_weights(NLAT)
    pct = (_legendre(LMAX, MMAX, cost) * wq[None, None, :])
    A = np.zeros((LMAX * MPAD, NLAT), np.float32)
    A.reshape(LMAX, MPAD, NLAT)[:, :MMAX, :] = pct
    n = np.arange(NLON)[:, None]
    m = np.arange(MPAD)[None, :]
    C = np.cos(2.0 * np.pi * m * n / NLON) * (2.0 * np.pi / NLON)
    C[:, MMAX:] = 0.0
    return A.astype(np.float32), C.astype(np.float32)


_A_NP, _C_NP = _build_consts()

_CX_NP = ((np.arange(NTR) * TR + (TR - 1) / 2.0) * SCALE).astype(np.float32)
_CY_NP = ((np.arange(NTC) * TC + (TC - 1) / 2.0) * SCALE - math.pi).astype(np.float32)
_HX = (TR - 1) / 2.0 * SCALE
_HY = (TC - 1) / 2.0 * SCALE


def _to_spherical(coords):
    x = coords[..., 0]
    y = coords[..., 1]
    z = coords[..., 2]
    r = jnp.sqrt(x * x + y * y + z * z)
    phi1 = jnp.arccos(jnp.clip(x / r, -1.0, 1.0))
    azn = jnp.sqrt(y * y + z * z)
    a = jnp.arccos(jnp.clip(y / azn, -1.0, 1.0))
    phi2 = a + (2.0 * math.pi - 2.0 * a) * (z < 0) - math.pi
    return r, jnp.stack([phi1, phi2], axis=-1)


def _tile_metadata(pts, rads):
    px = pts[:, :, 0][:, None, None, :]
    py = pts[:, :, 1][:, None, None, :]
    cx = jnp.asarray(_CX_NP)[None, :, None, None]
    cy = jnp.asarray(_CY_NP)[None, None, :, None]
    ax = jnp.abs(px - cx)
    ay = jnp.abs(py - cy)
    lbx = jnp.maximum(ax - _HX, 0.0)
    lby = jnp.maximum(ay - _HY, 0.0)
    lb = lbx * lbx + lby * lby
    ubx = ax + _HX
    uby = ay + _HY
    ub = ubx * ubx + uby * uby
    m1 = jnp.min(ub, axis=-1, keepdims=True)
    u2 = jnp.where(ub <= m1, jnp.inf, ub)
    m2 = jnp.min(u2, axis=-1, keepdims=True)
    u3 = jnp.where(u2 <= m2, jnp.inf, u2)
    ub3 = jnp.min(u3, axis=-1, keepdims=True)
    keep = lb <= ub3
    key = jnp.where(keep, 0, 1024) + jnp.arange(NPTS, dtype=jnp.int32)
    shape = key.shape
    bx = jnp.broadcast_to(px, shape)
    by = jnp.broadcast_to(py, shape)
    br = jnp.broadcast_to(rads[:, None, None, :], shape)
    _, spx, spy, sr = jax.lax.sort((key, bx, by, br), dimension=-1, num_keys=1)
    cnt = jnp.sum(keep, axis=-1, dtype=jnp.int32)
    cntu = jnp.minimum((cnt + UNROLL - 1) // UNROLL, NPTS // UNROLL)
    return spx, spy, sr, cntu


def _nn_interp_kernel(spx_ref, spy_ref, sr_ref, cnt_ref, out_ref):
    tr = pl.program_id(1)
    tc = pl.program_id(2)
    rowi = jax.lax.broadcasted_iota(jnp.int32, (TR, TC), 0).astype(jnp.float32)
    coli = jax.lax.broadcasted_iota(jnp.int32, (TR, TC), 1).astype(jnp.float32)
    gx = (rowi + (tr * TR).astype(jnp.float32)) * SCALE
    gy = (coli + (tc * TC).astype(jnp.float32)) * SCALE - math.pi

    big = jnp.full((TR, TC), 1e30, jnp.float32)
    zero = jnp.zeros((TR, TC), jnp.float32)

    def insert(c, carry):
        m1, m2, m3, r1, r2, r3 = carry
        px = spx_ref[0, 0, 0, c]
        py = spy_ref[0, 0, 0, c]
        rv = sr_ref[0, 0, 0, c]
        dx = gx - px
        dy = gy - py
        d = dx * dx + dy * dy
        c1 = d < m1
        nm1 = jnp.minimum(m1, d)
        t = jnp.maximum(m1, d)
        nr1 = jnp.where(c1, rv, r1)
        tr = jnp.where(c1, r1, rv)
        c2 = t < m2
        nm2 = jnp.minimum(m2, t)
        t2 = jnp.maximum(m2, t)
        nr2 = jnp.where(c2, tr, r2)
        tr2 = jnp.where(c2, r2, tr)
        c3 = t2 < m3
        nm3 = jnp.minimum(m3, t2)
        nr3 = jnp.where(c3, tr2, r3)
        return nm1, nm2, nm3, nr1, nr2, nr3

    def body(k, carry):
        base = k * UNROLL
        for u in range(UNROLL):
            carry = insert(base + u, carry)
        return carry

    m1, m2, m3, r1, r2, r3 = jax.lax.fori_loop(
        0, cnt_ref[0, 0, 0, 0], body, (big, big, big, zero, zero, zero))
    s = m1 + m2 + m3
    out_ref[0] = (r1 * m1 + r2 * m2 + r3 * m3) / s


def _sht_loss_kernel(interp_ref, c_ref, a_ref, out_ref):
    diff = interp_ref[0] - interp_ref[1]
    xc = jnp.dot(diff, c_ref[...], preferred_element_type=jnp.float32)
    y = jnp.dot(a_ref[...], xc, preferred_element_type=jnp.float32)
    row_m = jax.lax.broadcasted_iota(jnp.int32, y.shape, 0) % MPAD
    col_m = jax.lax.broadcasted_iota(jnp.int32, y.shape, 1)
    v = jnp.where(row_m == col_m, y, 0.0)
    out_ref[0, 0] = jnp.sum(v * v) / float(LMAX * MMAX)


def kernel(pred, target):
    rp, sp = _to_spherical(pred)
    rt, st = _to_spherical(target)
    pts = jnp.concatenate([sp, st], axis=0)
    rads = jnp.concatenate([rp, rt], axis=0)
    spx, spy, sr, cntu = _tile_metadata(pts, rads)
    smem_shape = (2, NTR * NTC, 1, NPTS)
    spx = spx.reshape(smem_shape)
    spy = spy.reshape(smem_shape)
    sr = sr.reshape(smem_shape)
    cntu = cntu.reshape(2, NTR * NTC, 1, 1)

    list_spec = pl.BlockSpec((1, 1, 1, NPTS),
                             lambda cl, tr, tc: (cl, tr * NTC + tc, 0, 0),
                             memory_space=pltpu.SMEM)
    interp = pl.pallas_call(
        _nn_interp_kernel,
        grid=(2, NTR, NTC),
        in_specs=[
            list_spec,
            list_spec,
            list_spec,
            pl.BlockSpec((1, 1, 1, 1),
                         lambda cl, tr, tc: (cl, tr * NTC + tc, 0, 0),
                         memory_space=pltpu.SMEM),
        ],
        out_specs=pl.BlockSpec((1, TR, TC), lambda cl, tr, tc: (cl, tr, tc)),
        out_shape=jax.ShapeDtypeStruct((2, NLAT, NLON), jnp.float32),
        compiler_params=pltpu.CompilerParams(
            dimension_semantics=("arbitrary", "arbitrary", "arbitrary")),
    )(spx, spy, sr, cntu)

    loss = pl.pallas_call(
        _sht_loss_kernel,
        in_specs=[
            pl.BlockSpec(memory_space=pltpu.VMEM),
            pl.BlockSpec(memory_space=pltpu.VMEM),
            pl.BlockSpec(memory_space=pltpu.VMEM),
        ],
        out_specs=pl.BlockSpec(memory_space=pltpu.SMEM),
        out_shape=jax.ShapeDtypeStruct((1, 1), jnp.float32),
    )(interp, jnp.asarray(_C_NP), jnp.asarray(_A_NP))

    return loss[0, 0]


# speedup vs baseline: 19.7093x; 1.2739x over previous
import functools
import math

import jax
import jax.numpy as jnp
import numpy as np
from jax import lax
from jax.experimental import pallas as pl
from jax.experimental.pallas import tpu as pltpu
from jax.experimental.pallas import tpu_sc as plsc

NLAT = 512
NLON = 1024
LMAX = 50
MMAX = 50
MPAD = 64
NPTS = 512
TR = 32
TC = 128
NTR = NLAT // TR
NTC = NLON // TC
SCALE = math.pi / 512.0
UNROLL = 16


def _cc_weights(n):
    N = n - 1
    theta = np.pi * np.arange(n) / N
    w = np.zeros(n)
    v = np.ones(n - 2)
    if N % 2 == 0:
        w0 = 1.0 / (N * N - 1)
        for k in range(1, N // 2):
            v -= 2.0 * np.cos(2 * k * theta[1:-1]) / (4 * k * k - 1)
        v -= np.cos(N * theta[1:-1]) / (N * N - 1)
    else:
        w0 = 1.0 / (N * N)
        for k in range(1, (N - 1) // 2 + 1):
            v -= 2.0 * np.cos(2 * k * theta[1:-1]) / (4 * k * k - 1)
    w[0] = w0
    w[-1] = w0
    w[1:-1] = 2.0 * v / N
    return w


def _legendre(lmax, mmax, x):
    nx = x.shape[0]
    P = np.zeros((lmax, mmax, nx))
    P[0, 0] = np.sqrt(1.0 / (4.0 * np.pi))
    s = np.sqrt(np.maximum(0.0, 1.0 - x * x))
    for m in range(1, mmax):
        P[m, m] = -np.sqrt((2.0 * m + 1.0) / (2.0 * m)) * s * P[m - 1, m - 1]
    for m in range(0, mmax):
        if m + 1 < lmax:
            P[m + 1, m] = np.sqrt(2.0 * m + 3.0) * x * P[m, m]
        for l in range(m + 2, lmax):
            a = np.sqrt((4.0 * l * l - 1.0) / (l * l - m * m))
            b = np.sqrt(((l - 1.0) ** 2 - m * m) / (4.0 * (l - 1.0) ** 2 - 1.0))
            P[l, m] = a * (x * P[l - 1, m] - b * P[l - 2, m])
    return P


def _build_consts():
    theta = np.pi * np.arange(NLAT) / (NLAT - 1)
    cost = np.cos(theta)
    wq = _cc_weights(NLAT)
    pct = (_legendre(LMAX, MMAX, cost) * wq[None, None, :])
    A = np.zeros((LMAX * MPAD, NLAT), np.float32)
    A.reshape(LMAX, MPAD, NLAT)[:, :MMAX, :] = pct
    n = np.arange(NLON)[:, None]
    m = np.arange(MPAD)[None, :]
    C = np.cos(2.0 * np.pi * m * n / NLON) * (2.0 * np.pi / NLON)
    C[:, MMAX:] = 0.0
    return A.astype(np.float32), C.astype(np.float32)


_A_NP, _C_NP = _build_consts()

_CX_NP = ((np.arange(NTR) * TR + (TR - 1) / 2.0) * SCALE).astype(np.float32)
_CY_NP = ((np.arange(NTC) * TC + (TC - 1) / 2.0) * SCALE - math.pi).astype(np.float32)
_HX = (TR - 1) / 2.0 * SCALE
_HY = (TC - 1) / 2.0 * SCALE


def _to_spherical(coords):
    x = coords[..., 0]
    y = coords[..., 1]
    z = coords[..., 2]
    r = jnp.sqrt(x * x + y * y + z * z)
    phi1 = jnp.arccos(jnp.clip(x / r, -1.0, 1.0))
    azn = jnp.sqrt(y * y + z * z)
    a = jnp.arccos(jnp.clip(y / azn, -1.0, 1.0))
    phi2 = a + (2.0 * math.pi - 2.0 * a) * (z < 0) - math.pi
    return r, jnp.stack([phi1, phi2], axis=-1)


def _tile_metadata(pts, rads):
    px = pts[:, :, 0][:, None, None, :]
    py = pts[:, :, 1][:, None, None, :]
    cx = jnp.asarray(_CX_NP)[None, :, None, None]
    cy = jnp.asarray(_CY_NP)[None, None, :, None]
    ax = jnp.abs(px - cx)
    ay = jnp.abs(py - cy)
    lbx = jnp.maximum(ax - _HX, 0.0)
    lby = jnp.maximum(ay - _HY, 0.0)
    lb = lbx * lbx + lby * lby
    ubx = ax + _HX
    uby = ay + _HY
    ub = ubx * ubx + uby * uby
    m1 = jnp.min(ub, axis=-1, keepdims=True)
    u2 = jnp.where(ub <= m1, jnp.inf, ub)
    m2 = jnp.min(u2, axis=-1, keepdims=True)
    u3 = jnp.where(u2 <= m2, jnp.inf, u2)
    ub3 = jnp.min(u3, axis=-1, keepdims=True)
    delta = (lb - ub3).reshape(2 * NTR * NTC, NPTS)
    return delta


NTASK = 2 * 16 * 8
NWORK = 32
TASKS_PER_W = NTASK // NWORK
NVEC = NPTS // 16


def _sc_compact(delta, pxf, pyf, prf):
    mesh = plsc.VectorSubcoreMesh(core_axis_name="c", subcore_axis_name="s")

    @functools.partial(
        pl.kernel,
        out_type=[
            jax.ShapeDtypeStruct((NTASK, NPTS), jnp.float32),
            jax.ShapeDtypeStruct((NTASK, NPTS), jnp.float32),
            jax.ShapeDtypeStruct((NTASK, NPTS), jnp.float32),
            jax.ShapeDtypeStruct((NTASK, 16), jnp.int32),
        ],
        mesh=mesh,
        compiler_params=pltpu.CompilerParams(needs_layout_passes=False),
        scratch_types=[
            pltpu.VMEM((NPTS,), jnp.float32),
            pltpu.VMEM((2 * NPTS,), jnp.float32),
            pltpu.VMEM((2 * NPTS,), jnp.float32),
            pltpu.VMEM((2 * NPTS,), jnp.float32),
            pltpu.VMEM((NPTS,), jnp.float32),
            pltpu.VMEM((NPTS,), jnp.float32),
            pltpu.VMEM((NPTS,), jnp.float32),
            pltpu.VMEM((16,), jnp.int32),
        ],
    )
    def k(delta_hbm, px_hbm, py_hbm, pr_hbm,
          opx_hbm, opy_hbm, opr_hbm, ocnt_hbm,
          dbuf, pxbuf, pybuf, prbuf, obx, oby, obr, cntbuf):
        wid = lax.axis_index("s") * 2 + lax.axis_index("c")
        pltpu.sync_copy(px_hbm, pxbuf)
        pltpu.sync_copy(py_hbm, pybuf)
        pltpu.sync_copy(pr_hbm, prbuf)
        zf = jnp.zeros((16,), jnp.float32)
        dummy = jnp.full((16,), 1e6, jnp.float32)
        onei = jnp.ones((16,), jnp.int32)
        zeroi = jnp.zeros((16,), jnp.int32)
        ids = lax.iota(jnp.int32, 16)
        last = jnp.full((16,), 15, jnp.int32)

        def scan16(s):
            for sh in (1, 2, 4, 8):
                shv = jnp.full((16,), sh, jnp.int32)
                g = jnp.take(s, jnp.maximum(ids - shv, zeroi))
                s = s + jnp.where(ids >= shv, g, zeroi)
            return s

        def task(i, carry):
            t = wid * TASKS_PER_W + i
            cl = t // (NTASK // 2)
            base = cl * NPTS
            pltpu.sync_copy(delta_hbm.at[t], dbuf)
            for j in range(NVEC):
                sl = pl.ds(j * 16, 16)
                obx[sl] = dummy
                oby[sl] = zf
                obr[sl] = zf
            cnt = jnp.zeros((16,), jnp.int32)
            for j in range(NVEC):
                sl = pl.ds(j * 16, 16)
                psl = pl.ds(base + j * 16, 16)
                m = dbuf[sl] <= zf
                incl = scan16(jnp.where(m, onei, zeroi))
                pos = cnt + incl - onei
                plsc.store_scatter(obx, [pos], pxbuf[psl], mask=m)
                plsc.store_scatter(oby, [pos], pybuf[psl], mask=m)
                plsc.store_scatter(obr, [pos], prbuf[psl], mask=m)
                cnt = cnt + jnp.take(incl, last)
            cntbuf[pl.ds(0, 16)] = cnt
            pltpu.sync_copy(obx, opx_hbm.at[t])
            pltpu.sync_copy(oby, opy_hbm.at[t])
            pltpu.sync_copy(obr, opr_hbm.at[t])
            pltpu.sync_copy(cntbuf, ocnt_hbm.at[t])
            return carry

        lax.fori_loop(0, TASKS_PER_W, task, 0)

    return k(delta, pxf, pyf, prf)


def _nn_interp_kernel(spx_ref, spy_ref, sr_ref, cnt_ref, out_ref):
    tr = pl.program_id(1)
    tc = pl.program_id(2)
    rowi = jax.lax.broadcasted_iota(jnp.int32, (TR, TC), 0).astype(jnp.float32)
    coli = jax.lax.broadcasted_iota(jnp.int32, (TR, TC), 1).astype(jnp.float32)
    gx = (rowi + (tr * TR).astype(jnp.float32)) * SCALE
    gy = (coli + (tc * TC).astype(jnp.float32)) * SCALE - math.pi

    big = jnp.full((TR, TC), 1e30, jnp.float32)
    zero = jnp.zeros((TR, TC), jnp.float32)

    def insert(c, carry):
        m1, m2, m3, r1, r2, r3 = carry
        px = spx_ref[0, 0, 0, c]
        py = spy_ref[0, 0, 0, c]
        rv = sr_ref[0, 0, 0, c]
        dx = gx - px
        dy = gy - py
        d = dx * dx + dy * dy
        c1 = d < m1
        nm1 = jnp.minimum(m1, d)
        t = jnp.maximum(m1, d)
        nr1 = jnp.where(c1, rv, r1)
        tr = jnp.where(c1, r1, rv)
        c2 = t < m2
        nm2 = jnp.minimum(m2, t)
        t2 = jnp.maximum(m2, t)
        nr2 = jnp.where(c2, tr, r2)
        tr2 = jnp.where(c2, r2, tr)
        c3 = t2 < m3
        nm3 = jnp.minimum(m3, t2)
        nr3 = jnp.where(c3, tr2, r3)
        return nm1, nm2, nm3, nr1, nr2, nr3

    def body(k, carry):
        base = k * UNROLL
        for u in range(UNROLL):
            carry = insert(base + u, carry)
        return carry

    m1, m2, m3, r1, r2, r3 = jax.lax.fori_loop(
        0, cnt_ref[0, 0, 0, 0], body, (big, big, big, zero, zero, zero))
    s = m1 + m2 + m3
    out_ref[0] = (r1 * m1 + r2 * m2 + r3 * m3) / s


def _sht_loss_kernel(interp_ref, c_ref, a_ref, out_ref):
    diff = interp_ref[0] - interp_ref[1]
    xc = jnp.dot(diff, c_ref[...], preferred_element_type=jnp.float32)
    y = jnp.dot(a_ref[...], xc, preferred_element_type=jnp.float32)
    row_m = jax.lax.broadcasted_iota(jnp.int32, y.shape, 0) % MPAD
    col_m = jax.lax.broadcasted_iota(jnp.int32, y.shape, 1)
    v = jnp.where(row_m == col_m, y, 0.0)
    out_ref[0, 0] = jnp.sum(v * v) / float(LMAX * MMAX)


def kernel(pred, target):
    rp, sp = _to_spherical(pred)
    rt, st = _to_spherical(target)
    pts = jnp.concatenate([sp, st], axis=0)
    rads = jnp.concatenate([rp, rt], axis=0)
    delta = _tile_metadata(pts, rads)
    spx, spy, sr, ocnt = _sc_compact(
        delta, pts[:, :, 0].reshape(-1), pts[:, :, 1].reshape(-1),
        rads.reshape(-1))
    cnt = ocnt[:, 0]
    cntu = jnp.minimum((cnt + UNROLL - 1) // UNROLL, NPTS // UNROLL)
    smem_shape = (2, NTR * NTC, 1, NPTS)
    spx = spx.reshape(smem_shape)
    spy = spy.reshape(smem_shape)
    sr = sr.reshape(smem_shape)
    cntu = cntu.reshape(2, NTR * NTC, 1, 1)

    list_spec = pl.BlockSpec((1, 1, 1, NPTS),
                             lambda cl, tr, tc: (cl, tr * NTC + tc, 0, 0),
                             memory_space=pltpu.SMEM)
    interp = pl.pallas_call(
        _nn_interp_kernel,
        grid=(2, NTR, NTC),
        in_specs=[
            list_spec,
            list_spec,
            list_spec,
            pl.BlockSpec((1, 1, 1, 1),
                         lambda cl, tr, tc: (cl, tr * NTC + tc, 0, 0),
                         memory_space=pltpu.SMEM),
        ],
        out_specs=pl.BlockSpec((1, TR, TC), lambda cl, tr, tc: (cl, tr, tc)),
        out_shape=jax.ShapeDtypeStruct((2, NLAT, NLON), jnp.float32),
        compiler_params=pltpu.CompilerParams(
            dimension_semantics=("arbitrary", "arbitrary", "arbitrary")),
    )(spx, spy, sr, cntu)

    loss = pl.pallas_call(
        _sht_loss_kernel,
        in_specs=[
            pl.BlockSpec(memory_space=pltpu.VMEM),
            pl.BlockSpec(memory_space=pltpu.VMEM),
            pl.BlockSpec(memory_space=pltpu.VMEM),
        ],
        out_specs=pl.BlockSpec(memory_space=pltpu.SMEM),
        out_shape=jax.ShapeDtypeStruct((1, 1), jnp.float32),
    )(interp, jnp.asarray(_C_NP), jnp.asarray(_A_NP))

    return loss[0, 0]


# speedup vs baseline: 21.4351x; 1.0876x over previous
import functools
import math

import jax
import jax.numpy as jnp
import numpy as np
from jax import lax
from jax.experimental import pallas as pl
from jax.experimental.pallas import tpu as pltpu
from jax.experimental.pallas import tpu_sc as plsc

NLAT = 512
NLON = 1024
LMAX = 50
MMAX = 50
MPAD = 64
NPTS = 512
TR = 64
TC = 64
NTR = NLAT // TR
NTC = NLON // TC
SCALE = math.pi / 512.0
UNROLL = 16


def _cc_weights(n):
    N = n - 1
    theta = np.pi * np.arange(n) / N
    w = np.zeros(n)
    v = np.ones(n - 2)
    if N % 2 == 0:
        w0 = 1.0 / (N * N - 1)
        for k in range(1, N // 2):
            v -= 2.0 * np.cos(2 * k * theta[1:-1]) / (4 * k * k - 1)
        v -= np.cos(N * theta[1:-1]) / (N * N - 1)
    else:
        w0 = 1.0 / (N * N)
        for k in range(1, (N - 1) // 2 + 1):
            v -= 2.0 * np.cos(2 * k * theta[1:-1]) / (4 * k * k - 1)
    w[0] = w0
    w[-1] = w0
    w[1:-1] = 2.0 * v / N
    return w


def _legendre(lmax, mmax, x):
    nx = x.shape[0]
    P = np.zeros((lmax, mmax, nx))
    P[0, 0] = np.sqrt(1.0 / (4.0 * np.pi))
    s = np.sqrt(np.maximum(0.0, 1.0 - x * x))
    for m in range(1, mmax):
        P[m, m] = -np.sqrt((2.0 * m + 1.0) / (2.0 * m)) * s * P[m - 1, m - 1]
    for m in range(0, mmax):
        if m + 1 < lmax:
            P[m + 1, m] = np.sqrt(2.0 * m + 3.0) * x * P[m, m]
        for l in range(m + 2, lmax):
            a = np.sqrt((4.0 * l * l - 1.0) / (l * l - m * m))
            b = np.sqrt(((l - 1.0) ** 2 - m * m) / (4.0 * (l - 1.0) ** 2 - 1.0))
            P[l, m] = a * (x * P[l - 1, m] - b * P[l - 2, m])
    return P


def _build_consts():
    theta = np.pi * np.arange(NLAT) / (NLAT - 1)
    cost = np.cos(theta)
    wq = _cc_weights(NLAT)
    pct = (_legendre(LMAX, MMAX, cost) * wq[None, None, :])
    A = np.zeros((LMAX * MPAD, NLAT), np.float32)
    A.reshape(LMAX, MPAD, NLAT)[:, :MMAX, :] = pct
    n = np.arange(NLON)[:, None]
    m = np.arange(MPAD)[None, :]
    C = np.cos(2.0 * np.pi * m * n / NLON) * (2.0 * np.pi / NLON)
    C[:, MMAX:] = 0.0
    cols0 = (np.arange(NLAT // 64)[:, None] * 64 + np.arange(32)[None, :]).ravel()
    A0 = A[:, cols0]
    A1 = A[:, cols0 + 32]
    return (A0.astype(np.float32), A1.astype(np.float32),
            C.reshape(NLON // 64, 64, MPAD).astype(np.float32))


_A0_NP, _A1_NP, _C_NP = _build_consts()

_CX_NP = ((np.arange(NTR) * TR + (TR - 1) / 2.0) * SCALE).astype(np.float32)
_CY_NP = ((np.arange(NTC) * TC + (TC - 1) / 2.0) * SCALE - math.pi).astype(np.float32)
_HX = (TR - 1) / 2.0 * SCALE
_HY = (TC - 1) / 2.0 * SCALE


def _to_spherical(coords):
    x = coords[..., 0]
    y = coords[..., 1]
    z = coords[..., 2]
    r = jnp.sqrt(x * x + y * y + z * z)
    phi1 = jnp.arccos(jnp.clip(x / r, -1.0, 1.0))
    azn = jnp.sqrt(y * y + z * z)
    a = jnp.arccos(jnp.clip(y / azn, -1.0, 1.0))
    phi2 = a + (2.0 * math.pi - 2.0 * a) * (z < 0) - math.pi
    return r, jnp.stack([phi1, phi2], axis=-1)


def _tile_metadata(pts, rads):
    px = pts[:, :, 0][:, None, None, :]
    py = pts[:, :, 1][:, None, None, :]
    cx = jnp.asarray(_CX_NP)[None, :, None, None]
    cy = jnp.asarray(_CY_NP)[None, None, :, None]
    ax = jnp.abs(px - cx)
    ay = jnp.abs(py - cy)
    lbx = jnp.maximum(ax - _HX, 0.0)
    lby = jnp.maximum(ay - _HY, 0.0)
    lb = lbx * lbx + lby * lby
    ubx = ax + _HX
    uby = ay + _HY
    ub = ubx * ubx + uby * uby
    m1 = jnp.min(ub, axis=-1, keepdims=True)
    u2 = jnp.where(ub <= m1, jnp.inf, ub)
    m2 = jnp.min(u2, axis=-1, keepdims=True)
    u3 = jnp.where(u2 <= m2, jnp.inf, u2)
    ub3 = jnp.min(u3, axis=-1, keepdims=True)
    delta = (lb - ub3).reshape(2 * NTR * NTC, NPTS)
    return delta


NTASK = 2 * NTR * NTC
NWORK = 32
TASKS_PER_W = NTASK // NWORK
NVEC = NPTS // 16


def _sc_compact(delta, pxf, pyf, prf):
    mesh = plsc.VectorSubcoreMesh(core_axis_name="c", subcore_axis_name="s")

    @functools.partial(
        pl.kernel,
        out_type=[
            jax.ShapeDtypeStruct((NTASK, NPTS), jnp.float32),
            jax.ShapeDtypeStruct((NTASK, NPTS), jnp.float32),
            jax.ShapeDtypeStruct((NTASK, NPTS), jnp.float32),
            jax.ShapeDtypeStruct((NTASK, 16), jnp.int32),
        ],
        mesh=mesh,
        compiler_params=pltpu.CompilerParams(needs_layout_passes=False),
        scratch_types=[
            pltpu.VMEM((NPTS,), jnp.float32),
            pltpu.VMEM((2 * NPTS,), jnp.float32),
            pltpu.VMEM((2 * NPTS,), jnp.float32),
            pltpu.VMEM((2 * NPTS,), jnp.float32),
            pltpu.VMEM((NPTS,), jnp.float32),
            pltpu.VMEM((NPTS,), jnp.float32),
            pltpu.VMEM((NPTS,), jnp.float32),
            pltpu.VMEM((16,), jnp.int32),
        ],
    )
    def k(delta_hbm, px_hbm, py_hbm, pr_hbm,
          opx_hbm, opy_hbm, opr_hbm, ocnt_hbm,
          dbuf, pxbuf, pybuf, prbuf, obx, oby, obr, cntbuf):
        wid = lax.axis_index("s") * 2 + lax.axis_index("c")
        pltpu.sync_copy(px_hbm, pxbuf)
        pltpu.sync_copy(py_hbm, pybuf)
        pltpu.sync_copy(pr_hbm, prbuf)
        zf = jnp.zeros((16,), jnp.float32)
        dummy = jnp.full((16,), 1e6, jnp.float32)
        onei = jnp.ones((16,), jnp.int32)
        zeroi = jnp.zeros((16,), jnp.int32)
        ids = lax.iota(jnp.int32, 16)
        last = jnp.full((16,), 15, jnp.int32)

        def scan16(s):
            for sh in (1, 2, 4, 8):
                shv = jnp.full((16,), sh, jnp.int32)
                g = jnp.take(s, jnp.maximum(ids - shv, zeroi))
                s = s + jnp.where(ids >= shv, g, zeroi)
            return s

        def task(i, carry):
            t = wid * TASKS_PER_W + i
            cl = t // (NTASK // 2)
            base = cl * NPTS
            pltpu.sync_copy(delta_hbm.at[t], dbuf)
            for j in range(NVEC):
                sl = pl.ds(j * 16, 16)
                obx[sl] = dummy
                oby[sl] = zf
                obr[sl] = zf
            cnt = jnp.zeros((16,), jnp.int32)
            for j in range(NVEC):
                sl = pl.ds(j * 16, 16)
                psl = pl.ds(base + j * 16, 16)
                m = dbuf[sl] <= zf
                incl = scan16(jnp.where(m, onei, zeroi))
                pos = cnt + incl - onei
                plsc.store_scatter(obx, [pos], pxbuf[psl], mask=m)
                plsc.store_scatter(oby, [pos], pybuf[psl], mask=m)
                plsc.store_scatter(obr, [pos], prbuf[psl], mask=m)
                cnt = cnt + jnp.take(incl, last)
            cntbuf[pl.ds(0, 16)] = cnt
            pltpu.sync_copy(obx, opx_hbm.at[t])
            pltpu.sync_copy(oby, opy_hbm.at[t])
            pltpu.sync_copy(obr, opr_hbm.at[t])
            pltpu.sync_copy(cntbuf, ocnt_hbm.at[t])
            return carry

        lax.fori_loop(0, TASKS_PER_W, task, 0)

    return k(delta, pxf, pyf, prf)


def _nn_interp_kernel(spx_ref, spy_ref, sr_ref, cnt_ref, out_ref):
    rr = pl.program_id(1)
    cc = pl.program_id(2)
    si = jax.lax.broadcasted_iota(jnp.int32, (32, 128), 0)
    li = jax.lax.broadcasted_iota(jnp.int32, (32, 128), 1)
    hoff = jnp.where(li >= 64, 32, 0)
    gx = (rr * TR + hoff + si).astype(jnp.float32) * SCALE
    gy = (cc * TC + (li & 63)).astype(jnp.float32) * SCALE - math.pi

    big = jnp.full((32, 128), 1e30, jnp.float32)
    zero = jnp.zeros((32, 128), jnp.float32)

    def insert(c, carry):
        m1, m2, m3, r1, r2, r3 = carry
        px = spx_ref[0, 0, 0, c]
        py = spy_ref[0, 0, 0, c]
        rv = sr_ref[0, 0, 0, c]
        dx = gx - px
        dy = gy - py
        d = dx * dx + dy * dy
        c1 = d < m1
        nm1 = jnp.minimum(m1, d)
        t = jnp.maximum(m1, d)
        nr1 = jnp.where(c1, rv, r1)
        tr = jnp.where(c1, r1, rv)
        c2 = t < m2
        nm2 = jnp.minimum(m2, t)
        t2 = jnp.maximum(m2, t)
        nr2 = jnp.where(c2, tr, r2)
        tr2 = jnp.where(c2, r2, tr)
        c3 = t2 < m3
        nm3 = jnp.minimum(m3, t2)
        nr3 = jnp.where(c3, tr2, r3)
        return nm1, nm2, nm3, nr1, nr2, nr3

    def body(k, carry):
        base = k * UNROLL
        for u in range(UNROLL):
            carry = insert(base + u, carry)
        return carry

    m1, m2, m3, r1, r2, r3 = jax.lax.fori_loop(
        0, cnt_ref[0, 0, 0, 0], body, (big, big, big, zero, zero, zero))
    s = m1 + m2 + m3
    out_ref[0, 0, 0] = (r1 * m1 + r2 * m2 + r3 * m3) / s


def _sht_loss_kernel(interp_ref, c_ref, a0_ref, a1_ref, out_ref):
    diff = interp_ref[0] - interp_ref[1]
    xc0 = jnp.zeros((NTR * 32, MPAD), jnp.float32)
    xc1 = jnp.zeros((NTR * 32, MPAD), jnp.float32)
    for cb in range(NTC):
        piece = diff[:, cb]
        p0 = piece[:, :, :64].reshape(NTR * 32, 64)
        p1 = piece[:, :, 64:].reshape(NTR * 32, 64)
        cm = c_ref[cb]
        xc0 = xc0 + jnp.dot(p0, cm, preferred_element_type=jnp.float32)
        xc1 = xc1 + jnp.dot(p1, cm, preferred_element_type=jnp.float32)
    y = (jnp.dot(a0_ref[...], xc0, preferred_element_type=jnp.float32) +
         jnp.dot(a1_ref[...], xc1, preferred_element_type=jnp.float32))
    row_m = jax.lax.broadcasted_iota(jnp.int32, y.shape, 0) % MPAD
    col_m = jax.lax.broadcasted_iota(jnp.int32, y.shape, 1)
    v = jnp.where(row_m == col_m, y, 0.0)
    out_ref[0, 0] = jnp.sum(v * v) / float(LMAX * MMAX)


def kernel(pred, target):
    rp, sp = _to_spherical(pred)
    rt, st = _to_spherical(target)
    pts = jnp.concatenate([sp, st], axis=0)
    rads = jnp.concatenate([rp, rt], axis=0)
    delta = _tile_metadata(pts, rads)
    spx, spy, sr, ocnt = _sc_compact(
        delta, pts[:, :, 0].reshape(-1), pts[:, :, 1].reshape(-1),
        rads.reshape(-1))
    cnt = ocnt[:, 0]
    cntu = jnp.minimum((cnt + UNROLL - 1) // UNROLL, NPTS // UNROLL)
    smem_shape = (2, NTR * NTC, 1, NPTS)
    spx = spx.reshape(smem_shape)
    spy = spy.reshape(smem_shape)
    sr = sr.reshape(smem_shape)
    cntu = cntu.reshape(2, NTR * NTC, 1, 1)

    list_spec = pl.BlockSpec((1, 1, 1, NPTS),
                             lambda cl, tr, tc: (cl, tr * NTC + tc, 0, 0),
                             memory_space=pltpu.SMEM)
    interp = pl.pallas_call(
        _nn_interp_kernel,
        grid=(2, NTR, NTC),
        in_specs=[
            list_spec,
            list_spec,
            list_spec,
            pl.BlockSpec((1, 1, 1, 1),
                         lambda cl, tr, tc: (cl, tr * NTC + tc, 0, 0),
                         memory_space=pltpu.SMEM),
        ],
        out_specs=pl.BlockSpec((1, 1, 1, 32, 128),
                               lambda cl, tr, tc: (cl, tr, tc, 0, 0)),
        out_shape=jax.ShapeDtypeStruct((2, NTR, NTC, 32, 128), jnp.float32),
        compiler_params=pltpu.CompilerParams(
            dimension_semantics=("arbitrary", "arbitrary", "arbitrary")),
    )(spx, spy, sr, cntu)

    loss = pl.pallas_call(
        _sht_loss_kernel,
        in_specs=[
            pl.BlockSpec(memory_space=pltpu.VMEM),
            pl.BlockSpec(memory_space=pltpu.VMEM),
            pl.BlockSpec(memory_space=pltpu.VMEM),
            pl.BlockSpec(memory_space=pltpu.VMEM),
        ],
        out_specs=pl.BlockSpec(memory_space=pltpu.SMEM),
        out_shape=jax.ShapeDtypeStruct((1, 1), jnp.float32),
    )(interp, jnp.asarray(_C_NP), jnp.asarray(_A0_NP), jnp.asarray(_A1_NP))

    return loss[0, 0]
